# baseline (speedup 1.0000x reference)
import functools

import jax
import jax.numpy as jnp
from jax import lax
from jax.experimental import pallas as pl
from jax.experimental.pallas import tpu as pltpu
from jax.experimental.pallas import tpu_sc as plsc

NC, NS = 2, 16
NW = NC * NS
B = 16384
D = 64
BPW = B // NW
CHUNK = 128
NCHUNK = BPW // CHUNK
NENTITY_PAIRS = 500000

_mesh = plsc.VectorSubcoreMesh(core_axis_name="c", subcore_axis_name="s")


@functools.partial(
    pl.kernel,
    out_type=jax.ShapeDtypeStruct((B, 2 * D), jnp.float32),
    mesh=_mesh,
    scratch_types=[
        pltpu.VMEM((NCHUNK, CHUNK), jnp.int32),
        pltpu.VMEM((BPW,), jnp.int32),
        pltpu.VMEM((BPW, 2 * D), jnp.float32),
        pltpu.VMEM((2, CHUNK, 2 * D), jnp.float32),
        pltpu.SemaphoreType.DMA((NCHUNK,)),
        pltpu.SemaphoreType.DMA((2,)),
    ],
)
def _lookup(pair_hbm, off_hbm, table2_hbm, out_hbm,
            pidx_v, off_v, rows2_v, big_v, gsems, osems):
    wid = lax.axis_index("s") * NC + lax.axis_index("c")
    base = wid * BPW

    pltpu.sync_copy(pair_hbm.at[wid], pidx_v)
    pltpu.sync_copy(off_hbm.at[wid], off_v)

    gathers = []
    for j in range(NCHUNK):
        gathers.append(
            pltpu.async_copy(
                table2_hbm.at[pidx_v.at[j]],
                rows2_v.at[pl.ds(j * CHUNK, CHUNK)],
                gsems.at[j],
            )
        )

    zrow = jnp.zeros((16,), jnp.float32)
    outs = [None, None]
    for j in range(NCHUNK):
        slot = j % 2
        gathers[j].wait()
        if outs[slot] is not None:
            outs[slot].wait()

        def _merge_group(g, carry):
            g16 = g * 16
            offv = off_v[pl.ds(j * CHUNK + g16, 16)]
            for r in range(16):
                off = offv[r]
                row = j * CHUNK + g16 + r
                for c in range(D // 16):
                    big_v[slot, g16 + r, pl.ds(c * 16, 16)] = (
                        rows2_v[row, pl.ds(off + c * 16, 16)]
                    )
                    big_v[slot, g16 + r, pl.ds(D + c * 16, 16)] = zrow
            return carry

        lax.fori_loop(0, CHUNK // 16, _merge_group, 0)

        outs[slot] = pltpu.async_copy(
            big_v.at[slot],
            out_hbm.at[pl.ds(base + j * CHUNK, CHUNK)],
            osems.at[slot],
        )

    for o in outs:
        o.wait()


def kernel(entity_ids, entity_table):
    ids = entity_ids.astype(jnp.int32)
    pair = (ids >> 1).reshape(NW, NCHUNK, CHUNK)
    off = ((ids & 1) * D).reshape(NW, BPW)
    table2 = entity_table.reshape(NENTITY_PAIRS, 2 * D)
    return _lookup(pair, off, table2)


# speedup vs baseline: 2.3728x; 2.3728x over previous
import functools

import jax
import jax.numpy as jnp
from jax import lax
from jax.experimental import pallas as pl
from jax.experimental.pallas import tpu as pltpu
from jax.experimental.pallas import tpu_sc as plsc

NC, NS = 2, 16
NW = NC * NS
B = 16384
D = 64
BPW = B // NW
CHUNK = 32
NCHUNK = BPW // CHUNK
NBLOCKS = 125000

_mesh = plsc.VectorSubcoreMesh(core_axis_name="c", subcore_axis_name="s")


@functools.partial(
    pl.kernel,
    out_type=jax.ShapeDtypeStruct((B, 2 * D), jnp.float32),
    mesh=_mesh,
    scratch_types=[
        pltpu.VMEM((NCHUNK, CHUNK), jnp.int32),
        pltpu.VMEM((BPW,), jnp.int32),
        pltpu.VMEM((2, CHUNK, 8, D), jnp.float32),
        pltpu.VMEM((2, CHUNK, 2 * D), jnp.float32),
        pltpu.SemaphoreType.DMA((2,)),
        pltpu.SemaphoreType.DMA((2,)),
    ],
)
def _lookup(bidx_hbm, off_hbm, tblk_hbm, out_hbm,
            bidx_v, off_v, blocks_v, big_v, gsems, osems):
    wid = lax.axis_index("s") * NC + lax.axis_index("c")
    base = wid * BPW

    pltpu.sync_copy(bidx_hbm.at[wid], bidx_v)
    pltpu.sync_copy(off_hbm.at[wid], off_v)

    def _fire_chunk(j, slot):
        for g in range(CHUNK // 16):
            bv = bidx_v[j, pl.ds(g * 16, 16)]
            for r in range(16):
                pltpu.async_copy(
                    tblk_hbm.at[bv[r]],
                    blocks_v.at[slot, g * 16 + r],
                    gsems.at[slot],
                )

    _fire_chunk(0, 0)
    zrow = jnp.zeros((16,), jnp.float32)

    def _step(j, carry):
        slot = j % 2

        @pl.when(j + 1 < NCHUNK)
        def _():
            _fire_chunk(j + 1, 1 - slot)

        for i in range(CHUNK):
            pltpu.make_async_copy(
                tblk_hbm.at[0], blocks_v.at[slot, i], gsems.at[slot]
            ).wait()

        @pl.when(j >= 2)
        def _():
            pltpu.make_async_copy(
                big_v.at[slot], out_hbm.at[pl.ds(base, CHUNK)], osems.at[slot]
            ).wait()

        def _merge_group(g, c2):
            g16 = g * 16
            offv = off_v[pl.ds(j * CHUNK + g16, 16)]
            for r in range(16):
                sub = offv[r]
                for c in range(D // 16):
                    big_v[slot, g16 + r, pl.ds(c * 16, 16)] = (
                        blocks_v[slot, g16 + r, sub, pl.ds(c * 16, 16)]
                    )
                    big_v[slot, g16 + r, pl.ds(D + c * 16, 16)] = zrow
            return c2

        lax.fori_loop(0, CHUNK // 16, _merge_group, 0)

        pltpu.async_copy(
            big_v.at[slot],
            out_hbm.at[pl.ds(base + j * CHUNK, CHUNK)],
            osems.at[slot],
        )
        return carry

    lax.fori_loop(0, NCHUNK, _step, 0)

    for slot in range(2):
        pltpu.make_async_copy(
            big_v.at[slot], out_hbm.at[pl.ds(base, CHUNK)], osems.at[slot]
        ).wait()


def kernel(entity_ids, entity_table):
    ids = entity_ids.astype(jnp.int32)
    bidx = (ids >> 3).reshape(NW, NCHUNK, CHUNK)
    off = (ids & 7).reshape(NW, BPW)
    tblk = entity_table.reshape(NBLOCKS, 8, D)
    return _lookup(bidx, off, tblk)


# speedup vs baseline: 3.0468x; 1.2840x over previous
import functools

import jax
import jax.numpy as jnp
from jax import lax
from jax.experimental import pallas as pl
from jax.experimental.pallas import tpu as pltpu
from jax.experimental.pallas import tpu_sc as plsc

NC, NS = 2, 16
NW = NC * NS
B = 16384
D = 64
BPW = B // NW
NG = BPW // 16
KRING = 8
OCHUNK = 64

_mesh = plsc.VectorSubcoreMesh(core_axis_name="c", subcore_axis_name="s")


@functools.partial(
    pl.kernel,
    out_type=jax.ShapeDtypeStruct((B, 2 * D), jnp.float32),
    mesh=_mesh,
    scratch_types=[
        pltpu.VMEM((BPW,), jnp.int32),
        pltpu.VMEM((BPW,), jnp.int32),
        pltpu.VMEM((KRING, D, 128), jnp.float32),
        pltpu.VMEM((2, OCHUNK, 2 * D), jnp.float32),
        pltpu.SemaphoreType.DMA((KRING,)),
        pltpu.SemaphoreType.DMA((2,)),
    ],
    compiler_params=pltpu.CompilerParams(needs_layout_passes=False),
)
def _lookup(slab_hbm, lane_hbm, tblt_hbm, out_hbm,
            slab_v, lane_v, ring_v, big_v, gsems, osems):
    wid = lax.axis_index("s") * NC + lax.axis_index("c")
    base = wid * BPW

    pltpu.sync_copy(slab_hbm.at[wid], slab_v)
    pltpu.sync_copy(lane_hbm.at[wid], lane_v)

    def _fire(i, sl):
        col = pl.multiple_of(sl * 128, 128)
        pltpu.async_copy(
            tblt_hbm.at[:, pl.ds(col, 128)],
            ring_v.at[i % KRING],
            gsems.at[i % KRING],
        )

    slabv0 = slab_v[pl.ds(0, 16)]
    for r in range(KRING):
        _fire(r, slabv0[r])

    zrow = jnp.zeros((16,), jnp.float32)
    rows16 = lax.iota(jnp.int32, 16)

    def _step(g, slabv_pair):
        slabv, slabv_next = slabv_pair
        lanev = lane_v[pl.ds(g * 16, 16)]
        obank = (g // 4) % 2

        @pl.when(jnp.logical_and(g % 4 == 0, g >= 8))
        def _():
            pltpu.make_async_copy(
                big_v.at[obank], out_hbm.at[pl.ds(base, OCHUNK)], osems.at[obank]
            ).wait()

        for r in range(16):
            i = g * 16 + r
            slot = i % KRING
            pltpu.make_async_copy(
                tblt_hbm.at[:, pl.ds(0, 128)], ring_v.at[slot], gsems.at[slot]
            ).wait()

            lane = lanev[r]
            colv = jnp.full((16,), lane, jnp.int32)
            brow = (g % 4) * 16 + r
            for c in range(D // 16):
                vals = plsc.load_gather(
                    ring_v.at[slot], [rows16 + c * 16, colv]
                )
                big_v[obank, brow, pl.ds(c * 16, 16)] = vals
                big_v[obank, brow, pl.ds(D + c * 16, 16)] = zrow

            nslab = jnp.where(r + KRING < 16, slabv[(r + KRING) % 16],
                              slabv_next[(r + KRING) % 16])

            @pl.when(i + KRING < BPW)
            def _():
                _fire(i + KRING, nslab)

        @pl.when(g % 4 == 3)
        def _():
            pltpu.async_copy(
                big_v.at[obank],
                out_hbm.at[pl.ds(base + (g - 3) * 16, OCHUNK)],
                osems.at[obank],
            )

        slabv2 = slabv_next
        idx_next = jnp.minimum((g + 2) * 16, BPW - 16)
        slabv_next2 = slab_v[pl.ds(idx_next, 16)]
        return (slabv2, slabv_next2)

    slabv_1 = slab_v[pl.ds(16, 16)]
    lax.fori_loop(0, NG, _step, (slabv0, slabv_1))

    for obank in range(2):
        pltpu.make_async_copy(
            big_v.at[obank], out_hbm.at[pl.ds(base, OCHUNK)], osems.at[obank]
        ).wait()


def kernel(entity_ids, entity_table):
    ids = entity_ids.astype(jnp.int32)
    slab = (ids >> 7).reshape(NW, BPW)
    lane = (ids & 127).reshape(NW, BPW)
    tblt = entity_table.T
    return _lookup(slab, lane, tblt)


# speedup vs baseline: 3.0873x; 1.0133x over previous
import functools

import jax
import jax.numpy as jnp
from jax import lax
from jax.experimental import pallas as pl
from jax.experimental.pallas import tpu as pltpu
from jax.experimental.pallas import tpu_sc as plsc

NC, NS = 2, 16
NW = NC * NS
B = 16384
D = 64
BPW = B // NW
NG = BPW // 16
KRING = 12
OCHUNK = 64

_mesh = plsc.VectorSubcoreMesh(core_axis_name="c", subcore_axis_name="s")


@functools.partial(
    pl.kernel,
    out_type=jax.ShapeDtypeStruct((B, 2 * D), jnp.float32),
    mesh=_mesh,
    scratch_types=[
        pltpu.VMEM((BPW,), jnp.int32),
        pltpu.VMEM((BPW,), jnp.int32),
        pltpu.VMEM((KRING, D, 128), jnp.float32),
        pltpu.VMEM((2, OCHUNK, 2 * D), jnp.float32),
        pltpu.SemaphoreType.DMA((KRING,)),
        pltpu.SemaphoreType.DMA((2,)),
    ],
    compiler_params=pltpu.CompilerParams(needs_layout_passes=False),
)
def _lookup(slab_hbm, lane_hbm, tblt_hbm, out_hbm,
            slab_v, lane_v, ring_v, big_v, gsems, osems):
    wid = lax.axis_index("s") * NC + lax.axis_index("c")
    base = wid * BPW

    pltpu.sync_copy(slab_hbm.at[wid], slab_v)
    pltpu.sync_copy(lane_hbm.at[wid], lane_v)

    def _fire(i, sl):
        col = pl.multiple_of(sl * 128, 128)
        pltpu.async_copy(
            tblt_hbm.at[:, pl.ds(col, 128)],
            ring_v.at[i % KRING],
            gsems.at[i % KRING],
        )

    slabv0 = slab_v[pl.ds(0, 16)]
    for r in range(KRING):
        _fire(r, slabv0[r])

    zrow = jnp.zeros((16,), jnp.float32)
    rows16 = lax.iota(jnp.int32, 16)

    def _step(g, slabv_pair):
        slabv, slabv_next = slabv_pair
        lanev = lane_v[pl.ds(g * 16, 16)]
        obank = (g // 4) % 2

        @pl.when(jnp.logical_and(g % 4 == 0, g >= 8))
        def _():
            pltpu.make_async_copy(
                big_v.at[obank], out_hbm.at[pl.ds(base, OCHUNK)], osems.at[obank]
            ).wait()

        for r in range(16):
            i = g * 16 + r
            slot = i % KRING
            pltpu.make_async_copy(
                tblt_hbm.at[:, pl.ds(0, 128)], ring_v.at[slot], gsems.at[slot]
            ).wait()

            nslab = jnp.where(r + KRING < 16, slabv[(r + KRING) % 16],
                              slabv_next[(r + KRING) % 16])

            @pl.when(i + KRING < BPW)
            def _():
                _fire(i + KRING, nslab)

            lane = lanev[r]
            colv = jnp.full((16,), lane, jnp.int32)
            brow = (g % 4) * 16 + r
            for c in range(D // 16):
                vals = plsc.load_gather(
                    ring_v.at[slot], [rows16 + c * 16, colv]
                )
                big_v[obank, brow, pl.ds(c * 16, 16)] = vals
                big_v[obank, brow, pl.ds(D + c * 16, 16)] = zrow

        @pl.when(g % 4 == 3)
        def _():
            pltpu.async_copy(
                big_v.at[obank],
                out_hbm.at[pl.ds(base + (g - 3) * 16, OCHUNK)],
                osems.at[obank],
            )

        slabv2 = slabv_next
        idx_next = jnp.minimum((g + 2) * 16, BPW - 16)
        slabv_next2 = slab_v[pl.ds(idx_next, 16)]
        return (slabv2, slabv_next2)

    slabv_1 = slab_v[pl.ds(16, 16)]
    lax.fori_loop(0, NG, _step, (slabv0, slabv_1))

    for obank in range(2):
        pltpu.make_async_copy(
            big_v.at[obank], out_hbm.at[pl.ds(base, OCHUNK)], osems.at[obank]
        ).wait()


def kernel(entity_ids, entity_table):
    ids = entity_ids.astype(jnp.int32)
    slab = (ids >> 7).reshape(NW, BPW)
    lane = (ids & 127).reshape(NW, BPW)
    tblt = entity_table.T
    return _lookup(slab, lane, tblt)


# speedup vs baseline: 4.6842x; 1.5173x over previous
import functools

import jax
import jax.numpy as jnp
from jax import lax
from jax.experimental import pallas as pl
from jax.experimental.pallas import tpu as pltpu
from jax.experimental.pallas import tpu_sc as plsc

NC, NS = 2, 16
NW = NC * NS
B = 16384
D = 64
BPW = B // NW
NG = BPW // 16
KRING = 12
AHEAD = 8
OCHUNK = 64
NOCH = BPW // OCHUNK

_mesh = plsc.VectorSubcoreMesh(core_axis_name="c", subcore_axis_name="s")


@functools.partial(
    pl.kernel,
    out_type=jax.ShapeDtypeStruct((B, 2 * D), jnp.float32),
    mesh=_mesh,
    scratch_types=[
        pltpu.VMEM((BPW + 16,), jnp.int32),
        pltpu.VMEM((BPW,), jnp.int32),
        pltpu.VMEM((BPW + 16,), jnp.int32),
        pltpu.VMEM((BPW + 16,), jnp.int32),
        pltpu.VMEM((NOCH, OCHUNK), jnp.int32),
        pltpu.VMEM((KRING, D, 128), jnp.float32),
        pltpu.VMEM((2, OCHUNK, 2 * D), jnp.float32),
        pltpu.SemaphoreType.DMA((KRING,)),
        pltpu.SemaphoreType.DMA((2,)),
    ],
    compiler_params=pltpu.CompilerParams(needs_layout_passes=False),
)
def _lookup(slab_hbm, lane_hbm, new_hbm, slot_hbm, row_hbm, tblt_hbm, out_hbm,
            slab_v, lane_v, new_v, slot_v, row_v, ring_v, big_v, gsems, osems):
    wid = lax.axis_index("s") * NC + lax.axis_index("c")

    pltpu.sync_copy(slab_hbm.at[wid], slab_v.at[pl.ds(0, BPW)])
    pltpu.sync_copy(lane_hbm.at[wid], lane_v)
    pltpu.sync_copy(new_hbm.at[wid], new_v.at[pl.ds(0, BPW)])
    pltpu.sync_copy(slot_hbm.at[wid], slot_v.at[pl.ds(0, BPW)])
    pltpu.sync_copy(row_hbm.at[wid], row_v)

    def _fire(sl, slot):
        col = pl.multiple_of(sl * 128, 128)
        pltpu.async_copy(
            tblt_hbm.at[:, pl.ds(col, 128)], ring_v.at[slot], gsems.at[slot]
        )

    slab0 = slab_v[pl.ds(0, 16)]
    new0 = new_v[pl.ds(0, 16)]
    slot0 = slot_v[pl.ds(0, 16)]
    for r in range(AHEAD):
        @pl.when(new0[r] == 1)
        def _():
            _fire(slab0[r], slot0[r])

    zrow = jnp.zeros((16,), jnp.float32)
    rows16 = lax.iota(jnp.int32, 16)

    def _step(g, carry):
        g16 = g * 16
        lanev = lane_v[pl.ds(g16, 16)]
        newv = new_v[pl.ds(g16, 16)]
        slotv = slot_v[pl.ds(g16, 16)]
        slaba = slab_v[pl.ds(g16 + AHEAD, 16)]
        newa = new_v[pl.ds(g16 + AHEAD, 16)]
        slota = slot_v[pl.ds(g16 + AHEAD, 16)]
        obank = (g // 4) % 2

        @pl.when(jnp.logical_and(g % 4 == 0, g >= 8))
        def _():
            pltpu.make_async_copy(
                big_v.at[obank], out_hbm.at[row_v.at[0]], osems.at[obank]
            ).wait()

        for r in range(16):
            i = g16 + r
            slot = slotv[r]

            @pl.when(jnp.logical_and(newa[r] == 1, i + AHEAD < BPW))
            def _():
                _fire(slaba[r], slota[r])

            @pl.when(newv[r] == 1)
            def _():
                pltpu.make_async_copy(
                    tblt_hbm.at[:, pl.ds(0, 128)], ring_v.at[slot],
                    gsems.at[slot],
                ).wait()

            colv = jnp.full((16,), lanev[r], jnp.int32)
            brow = (g % 4) * 16 + r
            for c in range(D // 16):
                vals = plsc.load_gather(ring_v.at[slot], [rows16 + c * 16, colv])
                big_v[obank, brow, pl.ds(c * 16, 16)] = vals
                big_v[obank, brow, pl.ds(D + c * 16, 16)] = zrow

        @pl.when(g % 4 == 3)
        def _():
            pltpu.async_copy(
                big_v.at[obank],
                out_hbm.at[row_v.at[g // 4]],
                osems.at[obank],
            )

        return carry

    lax.fori_loop(0, NG, _step, 0)

    for obank in range(2):
        pltpu.make_async_copy(
            big_v.at[obank], out_hbm.at[row_v.at[0]], osems.at[obank]
        ).wait()


def kernel(entity_ids, entity_table):
    ids = entity_ids.astype(jnp.int32)
    slab_all = ids >> 7
    order = jnp.argsort(slab_all).astype(jnp.int32)
    ids_s = jnp.take(ids, order)
    slab2 = (ids_s >> 7).reshape(NW, BPW)
    lane = (ids_s & 127).reshape(NW, BPW)
    prev = jnp.concatenate([slab2[:, :1] - 1, slab2[:, :-1]], axis=1)
    new = (slab2 != prev).astype(jnp.int32)
    slot = (jnp.cumsum(new, axis=1) - 1) % KRING
    rows = order.reshape(NW, NOCH, OCHUNK)
    return _lookup(slab2, lane, new, slot.astype(jnp.int32), rows,
                   entity_table.T)
